# Initial kernel scaffold; baseline (speedup 1.0000x reference)
import jax
import jax.numpy as jnp
from jax.experimental import pallas as pl


def kernel(x, table):
    raise NotImplementedError("write your pallas kernel here")


# speedup vs baseline: 3.5443x; 3.5443x over previous
import functools

import jax
import jax.numpy as jnp
from jax import lax
from jax.experimental import pallas as pl
from jax.experimental.pallas import tpu as pltpu
from jax.experimental.pallas import tpu_sc as plsc

B, L, D = 4096, 200, 64
N = B * L
NW = 32
CH = 128
ROWS_PER_W = N // NW
NCH = ROWS_PER_W // CH

_mesh = plsc.VectorSubcoreMesh(core_axis_name="c", subcore_axis_name="s")


@functools.partial(
    pl.kernel,
    mesh=_mesh,
    out_type=jax.ShapeDtypeStruct((N, D), jnp.float32),
    scratch_types=[
        pltpu.VMEM((NCH, CH), jnp.int32),
        pltpu.VMEM((CH, D), jnp.float32),
        pltpu.SemaphoreType.DMA,
    ],
    compiler_params=pltpu.CompilerParams(use_tc_tiling_on_sc=False),
)
def _emb_lookup(x_hbm, table_hbm, out_hbm, idx_v, rows_v, sem):
    wid = lax.axis_index("s") * 2 + lax.axis_index("c")
    base_chunk = wid * NCH
    pltpu.sync_copy(x_hbm.at[pl.ds(base_chunk, NCH)], idx_v)

    def body(j, carry):
        pltpu.async_copy(table_hbm.at[idx_v.at[j]], rows_v, sem).wait()
        pltpu.sync_copy(rows_v, out_hbm.at[pl.ds((base_chunk + j) * CH, CH)])
        return carry

    lax.fori_loop(0, NCH, body, 0)


def kernel(x, table):
    xf = x.reshape(N // CH, CH)
    out = _emb_lookup(xf, table)
    return out.reshape(B, L, D)


# speedup vs baseline: 4.2621x; 1.2025x over previous
import functools

import jax
import jax.numpy as jnp
from jax import lax
from jax.experimental import pallas as pl
from jax.experimental.pallas import tpu as pltpu
from jax.experimental.pallas import tpu_sc as plsc

B, L, D = 4096, 200, 64
NW = 32
BW = B // NW
S0, S1 = 104, 96
NBUF = 4
NG = BW // NBUF

_mesh = plsc.VectorSubcoreMesh(core_axis_name="c", subcore_axis_name="s")


@functools.partial(
    pl.kernel,
    mesh=_mesh,
    out_type=jax.ShapeDtypeStruct((B, L, D), jnp.float32),
    scratch_types=[
        pltpu.VMEM((BW, L), jnp.int32),
        pltpu.VMEM((NBUF, L, D), jnp.float32),
        [pltpu.SemaphoreType.DMA] * NBUF,
    ],
    compiler_params=pltpu.CompilerParams(use_tc_tiling_on_sc=False),
)
def _emb_lookup(x_hbm, table_hbm, out_hbm, idx_v, rows_v, sems):
    wid = lax.axis_index("s") * 2 + lax.axis_index("c")
    b0 = wid * BW
    pltpu.sync_copy(x_hbm.at[pl.ds(b0, BW)], idx_v)

    def gather_start(bb, b):
        pltpu.make_async_copy(
            table_hbm.at[idx_v.at[bb, pl.ds(0, S0)]],
            rows_v.at[b, pl.ds(0, S0), :],
            sems[b],
        ).start()
        pltpu.make_async_copy(
            table_hbm.at[idx_v.at[bb, pl.ds(S0, S1)]],
            rows_v.at[b, pl.ds(S0, S1), :],
            sems[b],
        ).start()

    def gather_wait(b):
        pltpu.make_async_copy(
            table_hbm.at[idx_v.at[0, pl.ds(0, S0)]],
            rows_v.at[b, pl.ds(0, S0), :],
            sems[b],
        ).wait()
        pltpu.make_async_copy(
            table_hbm.at[idx_v.at[0, pl.ds(S0, S1)]],
            rows_v.at[b, pl.ds(S0, S1), :],
            sems[b],
        ).wait()

    for b in range(NBUF):
        gather_start(b, b)

    def body(g, carry):
        for b in range(NBUF):
            bb = g * NBUF + b
            gather_wait(b)
            pltpu.sync_copy(rows_v.at[b], out_hbm.at[b0 + bb])

            @pl.when(g < NG - 1)
            def _():
                gather_start(bb + NBUF, b)

        return carry

    lax.fori_loop(0, NG, body, 0)


def kernel(x, table):
    return _emb_lookup(x, table)
